# baseline (speedup 1.0000x reference)
import functools

import jax
import jax.numpy as jnp
from jax import lax
from jax.experimental import pallas as pl
from jax.experimental.pallas import tpu as pltpu
from jax.experimental.pallas import tpu_sc as plsc

T, D = 8192, 2048
NSEG, SEG = 8, 1024
WPS = 4
CPW = D // WPS
G = CPW // 16
TCH = 16
NCH = SEG // TCH
NBUF = 4


def _sc_ema(x, a, s):
    mesh = plsc.VectorSubcoreMesh(core_axis_name="c", subcore_axis_name="s")

    @functools.partial(
        pl.kernel,
        out_type=jax.ShapeDtypeStruct((T, D), jnp.float32),
        mesh=mesh,
        scratch_types=(
            [pltpu.VMEM((TCH, CPW), jnp.float32)] * NBUF
            + [pltpu.VMEM((TCH, CPW), jnp.float32)] * NBUF
            + [
                pltpu.VMEM((SEG,), jnp.float32),
                pltpu.VMEM((SEG,), jnp.float32),
            ]
            + [pltpu.SemaphoreType.DMA] * (2 * NBUF)
        ),
    )
    def body(x_hbm, a_hbm, s_hbm, out_hbm, *scr):
        xvs = scr[:NBUF]
        ovs = scr[NBUF:2 * NBUF]
        av, sv = scr[2 * NBUF], scr[2 * NBUF + 1]
        sis = scr[2 * NBUF + 2:3 * NBUF + 2]
        sos = scr[3 * NBUF + 2:4 * NBUF + 2]
        wid = lax.axis_index("s") * 2 + lax.axis_index("c")
        seg = wid // WPS
        c0 = (wid % WPS) * CPW
        t0 = seg * SEG
        pltpu.sync_copy(a_hbm.at[pl.ds(t0, SEG)], av)
        pltpu.sync_copy(s_hbm.at[pl.ds(t0, SEG)], sv)

        def in_slice(ch):
            return x_hbm.at[pl.ds(t0 + ch * TCH, TCH), pl.ds(c0, CPW)]

        def out_slice(ch):
            return out_hbm.at[pl.ds(t0 + ch * TCH, TCH), pl.ds(c0, CPW)]

        for k in range(NBUF - 1):
            pltpu.async_copy(in_slice(k), xvs[k], sis[k])

        def compute(ch, xv, ov, hs):
            tq = ch * TCH
            avq = av[pl.ds(tq, 16)]
            svq = sv[pl.ds(tq, 16)]
            hl = list(hs)
            for j in range(16):
                a_t = avq[j]
                s_t = svq[j]
                for g in range(G):
                    h = a_t * hl[g] + s_t * xv[j, pl.ds(g * 16, 16)]
                    hl[g] = h
                    ov[j, pl.ds(g * 16, 16)] = h
            return tuple(hl)

        def ring_body(i, hs):
            for b in range(NBUF):
                ch = NBUF * i + b
                pltpu.make_async_copy(in_slice(ch), xvs[b], sis[b]).wait()

                @pl.when(ch + NBUF - 1 < NCH)
                def _():
                    pltpu.async_copy(in_slice(ch + NBUF - 1),
                                     xvs[(b + NBUF - 1) % NBUF],
                                     sis[(b + NBUF - 1) % NBUF])

                @pl.when(ch >= NBUF)
                def _():
                    pltpu.make_async_copy(ovs[b], out_slice(ch - NBUF),
                                          sos[b]).wait()

                hs = compute(ch, xvs[b], ovs[b], hs)
                pltpu.async_copy(ovs[b], out_slice(ch), sos[b])
            return hs

        zeros = jnp.zeros((16,), jnp.float32)
        lax.fori_loop(0, NCH // NBUF, ring_body, (zeros,) * G, unroll=False)
        for b in range(NBUF):
            pltpu.make_async_copy(ovs[b], out_slice(NCH - NBUF + b),
                                  sos[b]).wait()

    return body(x, a, s)


def kernel(hidden_states, boundary_mask, boundary_prob, cu_seqlens):
    p = jnp.clip(boundary_prob[:, 1].astype(jnp.float32), 1e-4, 1.0 - 1e-4)
    a = jnp.where(boundary_mask, 1.0 - p, 1.0)
    s = jnp.where(boundary_mask, p, 0.0)
    return _sc_ema(hidden_states.astype(jnp.float32), a, s)
